# baseline (speedup 1.0000x reference)
import functools
import math

import jax
import jax.numpy as jnp
from jax import lax
from jax.experimental import pallas as pl
from jax.experimental.pallas import tpu as pltpu
from jax.experimental.pallas import tpu_sc as plsc

_SIZE = 100000
_PADDING_IDX = 0
_LABEL_SMOOTHING = 0.1
_SV = _LABEL_SMOOTHING / (_SIZE - 2)
_CONF = 1.0 - _LABEL_SMOOTHING
_C0 = (_SIZE - 2) * _SV * math.log(_SV) + _CONF * math.log(_CONF)

_N = 1024
_BLOCK_W = 4096
_ROW_BLOCKS = 2
_NUM_BLOCKS = pl.cdiv(_SIZE, _BLOCK_W)

_TILE_LIM = (_SIZE // 128) * 128
_LAST_FULL_TILE = _SIZE // 128 - 1

_NC = 2
_NS = 16
_NW = _NC * _NS

_S_TC = 512
_SC_ROWS = _N - _S_TC
_ROWS_PER_TEC = _SC_ROWS // _NW
_STRIPES_PER_TEC = _ROWS_PER_TEC // 8

_ROWS_PER_W = _N // _NW
_VECS_PER_W = _ROWS_PER_W // 16

_CW = 4096
_NFULL = _TILE_LIM // _CW
_CREM = _TILE_LIM - _NFULL * _CW


def _tc_body(x_ref, t_ref, g_ref, out_ref):
    k = pl.program_id(1)
    x = x_ref[...]
    t = t_ref[...]
    mask = (t != _PADDING_IDX).astype(jnp.float32)

    @pl.when(k == 0)
    def _first():
        s_rows = jnp.sum(x, axis=1, keepdims=True) - x[:, 0:1]
        partial = jnp.sum(s_rows * mask, axis=(0, 1), keepdims=True)
        cnt = jnp.sum(mask, axis=(0, 1), keepdims=True)
        g = g_ref[...]
        lanes16 = jax.lax.broadcasted_iota(jnp.int32, g.shape, 1)
        sel = (lanes16 == t % 16).astype(jnp.float32)
        o_t = jnp.sum(g * sel, axis=1, keepdims=True)
        mask_lo = mask * (t < _TILE_LIM).astype(jnp.float32)
        corr = jnp.sum(o_t * mask_lo, axis=(0, 1), keepdims=True)
        out_ref[0, 0] = _C0 * cnt - _SV * partial + (_SV - _CONF) * corr

    @pl.when((k != 0) & (k != _NUM_BLOCKS - 1))
    def _mid():
        s_rows = jnp.sum(x, axis=1, keepdims=True)
        partial = jnp.sum(s_rows * mask, axis=(0, 1), keepdims=True)
        out_ref[0, 0] = -_SV * partial

    @pl.when(k == _NUM_BLOCKS - 1)
    def _last():
        cols = (_NUM_BLOCKS - 1) * _BLOCK_W + jax.lax.broadcasted_iota(
            jnp.int32, x.shape, 1)
        xv = jnp.where(cols < _SIZE, x, 0.0)
        s_rows = jnp.sum(xv, axis=1, keepdims=True)
        partial = jnp.sum(s_rows * mask, axis=(0, 1), keepdims=True)
        sel_hi = ((cols == t) & (t >= _TILE_LIM)).astype(jnp.float32)
        o_t_hi = jnp.sum(xv * sel_hi, axis=1, keepdims=True)
        corr_hi = jnp.sum(o_t_hi * mask, axis=(0, 1), keepdims=True)
        out_ref[0, 0] = -_SV * partial + (_SV - _CONF) * corr_hi


def _tc_partials(output, t32, gathered):
    rb = _S_TC // _ROW_BLOCKS
    return pl.pallas_call(
        _tc_body,
        grid=(_ROW_BLOCKS, _NUM_BLOCKS),
        in_specs=[
            pl.BlockSpec((rb, _BLOCK_W), lambda r, k: (r, k)),
            pl.BlockSpec((rb, 1), lambda r, k: (r, 0)),
            pl.BlockSpec((rb, 16), lambda r, k: (r, 0)),
        ],
        out_specs=pl.BlockSpec((1, 1, 1, 1), lambda r, k: (r, k, 0, 0)),
        out_shape=jax.ShapeDtypeStruct((_ROW_BLOCKS, _NUM_BLOCKS, 1, 1),
                                       jnp.float32),
        compiler_params=pltpu.CompilerParams(
            dimension_semantics=("parallel", "parallel"),
        ),
    )(output, t32, gathered)


def _sc_gather_body(x_hbm, tgt_hbm, out_hbm, t_v, tile_v, obuf_v, sem):
    wid = lax.axis_index("s") * _NC + lax.axis_index("c")
    base = wid * _ROWS_PER_W
    pltpu.sync_copy(tgt_hbm.at[pl.ds(base, _ROWS_PER_W)], t_v)
    copies = []
    scalars = []
    for c in range(_VECS_PER_W):
        t16 = t_v[pl.ds(c * 16, 16)]
        for l in range(16):
            r = c * 16 + l
            t_r = t16[l]
            tile = jnp.minimum(lax.shift_right_logical(t_r, 7),
                               _LAST_FULL_TILE)
            col0 = pl.multiple_of(lax.shift_left(tile, 7), 128)
            row0 = pl.multiple_of(base + 8 * (r // 8), 8)
            scalars.append(t_r)
            copies.append(pltpu.async_copy(
                x_hbm.at[pl.ds(row0, 8), pl.ds(col0, 128)],
                tile_v.at[r], sem))
    for cp in copies:
        cp.wait()
    for r in range(_ROWS_PER_W):
        t_r = scalars[r]
        a = jnp.bitwise_and(t_r, 127) - jnp.bitwise_and(t_r, 15)
        obuf_v[r] = tile_v[r, r % 8, pl.ds(a, 16)]
    pltpu.sync_copy(obuf_v, out_hbm.at[pl.ds(base, _ROWS_PER_W)])


def _sc_gather(output, t32):
    tgt = t32.reshape(_N)
    mesh = plsc.VectorSubcoreMesh(core_axis_name="c", subcore_axis_name="s")
    f = functools.partial(
        pl.kernel,
        mesh=mesh,
        out_type=jax.ShapeDtypeStruct((_N, 16), jnp.float32),
        scratch_types=[
            pltpu.VMEM((_ROWS_PER_W,), jnp.int32),
            pltpu.VMEM((_ROWS_PER_W, 8, 128), jnp.float32),
            pltpu.VMEM((_ROWS_PER_W, 16), jnp.float32),
            pltpu.SemaphoreType.DMA,
        ],
    )(_sc_gather_body)
    return f(output, tgt)


def _sc_stream_body(x_hbm, out_hbm, buf_v, vout_v, sem):
    wid = lax.axis_index("s") * _NC + lax.axis_index("c")
    for j in range(_STRIPES_PER_TEC):
        row0 = pl.multiple_of(
            _S_TC + (wid * _STRIPES_PER_TEC + j) * 8, 8)
        accs = tuple(jnp.zeros((16,), jnp.float32) for _ in range(8))

        def chunk_body(i, accs):
            c0 = pl.multiple_of(i * _CW, 128)
            pltpu.async_copy(
                x_hbm.at[pl.ds(row0, 8), pl.ds(c0, _CW)], buf_v, sem
            ).wait()

            def vec_body(v, accs):
                o = v * 16
                return tuple(accs[s] + buf_v[s, pl.ds(o, 16)]
                             for s in range(8))

            return lax.fori_loop(0, _CW // 16, vec_body, accs)

        accs = lax.fori_loop(0, _NFULL, chunk_body, accs)

        c0 = pl.multiple_of(_NFULL * _CW, 128)
        pltpu.async_copy(
            x_hbm.at[pl.ds(row0, 8), pl.ds(c0, _CREM)],
            buf_v.at[:, pl.ds(0, _CREM)], sem).wait()

        def rem_body(v, accs):
            o = v * 16
            return tuple(accs[s] + buf_v[s, pl.ds(o, 16)] for s in range(8))

        accs = lax.fori_loop(0, _CREM // 16, rem_body, accs)
        for s in range(8):
            vout_v[j * 8 + s] = accs[s]

    pltpu.sync_copy(vout_v, out_hbm.at[pl.ds(wid * _ROWS_PER_TEC,
                                             _ROWS_PER_TEC)])


def _sc_stream(output):
    mesh = plsc.VectorSubcoreMesh(core_axis_name="c", subcore_axis_name="s")
    f = functools.partial(
        pl.kernel,
        mesh=mesh,
        out_type=jax.ShapeDtypeStruct((_SC_ROWS, 16), jnp.float32),
        scratch_types=[
            pltpu.VMEM((8, _CW), jnp.float32),
            pltpu.VMEM((_ROWS_PER_TEC, 16), jnp.float32),
            pltpu.SemaphoreType.DMA,
        ],
    )(_sc_stream_body)
    return f(output)


def _comb_body(scs_ref, tail_ref, col0_ref, g_ref, t_ref, out_ref):
    t = t_ref[...][_S_TC:, :]
    mask = (t != _PADDING_IDX).astype(jnp.float32)
    scs = jnp.sum(scs_ref[...], axis=1, keepdims=True)
    tail = tail_ref[...][_S_TC:, :]
    cols = _TILE_LIM + jax.lax.broadcasted_iota(jnp.int32, tail.shape, 1)
    tailv = jnp.where(cols < _SIZE, tail, 0.0)
    tail_sum = jnp.sum(tailv, axis=1, keepdims=True)
    o0 = col0_ref[...][_S_TC:, 0:1]
    g = g_ref[...][_S_TC:, :]
    lanes16 = jax.lax.broadcasted_iota(jnp.int32, g.shape, 1)
    sel = (lanes16 == t % 16).astype(jnp.float32)
    o_t_lo = (jnp.sum(g * sel, axis=1, keepdims=True)
              * (t < _TILE_LIM).astype(jnp.float32))
    sel_hi = (cols == t).astype(jnp.float32)
    o_t_hi = jnp.sum(tailv * sel_hi, axis=1, keepdims=True)
    o_t = o_t_lo + o_t_hi
    s_all = scs + tail_sum - o0 - o_t
    loss = _C0 - _SV * s_all - _CONF * o_t
    out_ref[...] = jnp.sum(loss * mask, axis=(0, 1), keepdims=True)


def _combine(scs, output, gathered, t32):
    return pl.pallas_call(
        _comb_body,
        grid=(1,),
        in_specs=[
            pl.BlockSpec((_SC_ROWS, 16), lambda k: (0, 0)),
            pl.BlockSpec((_N, 128), lambda k: (0, _TILE_LIM // 128)),
            pl.BlockSpec((_N, 128), lambda k: (0, 0)),
            pl.BlockSpec((_N, 16), lambda k: (0, 0)),
            pl.BlockSpec((_N, 1), lambda k: (0, 0)),
        ],
        out_specs=pl.BlockSpec((1, 1), lambda k: (0, 0)),
        out_shape=jax.ShapeDtypeStruct((1, 1), jnp.float32),
    )(scs, output, output, gathered, t32)


@jax.jit
def kernel(output, target):
    t32 = target.astype(jnp.int32)
    gathered = _sc_gather(output, t32)
    scs = _sc_stream(output)
    tc = _tc_partials(output, t32, gathered)
    comb = _combine(scs, output, gathered, t32)
    return jnp.sum(tc) + comb[0, 0]


# speedup vs baseline: 1.0009x; 1.0009x over previous
import functools
import math

import jax
import jax.numpy as jnp
from jax import lax
from jax.experimental import pallas as pl
from jax.experimental.pallas import tpu as pltpu
from jax.experimental.pallas import tpu_sc as plsc

_SIZE = 100000
_PADDING_IDX = 0
_LABEL_SMOOTHING = 0.1
_SV = _LABEL_SMOOTHING / (_SIZE - 2)
_CONF = 1.0 - _LABEL_SMOOTHING
_C0 = (_SIZE - 2) * _SV * math.log(_SV) + _CONF * math.log(_CONF)

_N = 1024
_BLOCK_W = 4096
_ROW_BLOCKS = 2
_NUM_BLOCKS = pl.cdiv(_SIZE, _BLOCK_W)

_TILE_LIM = (_SIZE // 128) * 128
_LAST_FULL_TILE = _SIZE // 128 - 1

_NC = 2
_NS = 16
_NW = _NC * _NS

_S_TC = 512
_SC_ROWS = _N - _S_TC
_ROWS_PER_TEC = _SC_ROWS // _NW
_STRIPES_PER_TEC = _ROWS_PER_TEC // 8

_ROWS_PER_W = _N // _NW
_VECS_PER_W = _ROWS_PER_W // 16

_CW = 4096
_NFULL = _TILE_LIM // _CW
_CREM = _TILE_LIM - _NFULL * _CW


def _tc_body(x_ref, t_ref, out_ref):
    k = pl.program_id(1)
    x = x_ref[...]
    t = t_ref[...]
    mask = (t != _PADDING_IDX).astype(jnp.float32)

    @pl.when(k == 0)
    def _first():
        s_rows = jnp.sum(x, axis=1, keepdims=True) - x[:, 0:1]
        partial = jnp.sum(s_rows * mask, axis=(0, 1), keepdims=True)
        cnt = jnp.sum(mask, axis=(0, 1), keepdims=True)
        out_ref[0, 0] = _C0 * cnt - _SV * partial

    @pl.when((k != 0) & (k != _NUM_BLOCKS - 1))
    def _mid():
        s_rows = jnp.sum(x, axis=1, keepdims=True)
        partial = jnp.sum(s_rows * mask, axis=(0, 1), keepdims=True)
        out_ref[0, 0] = -_SV * partial

    @pl.when(k == _NUM_BLOCKS - 1)
    def _last():
        cols = (_NUM_BLOCKS - 1) * _BLOCK_W + jax.lax.broadcasted_iota(
            jnp.int32, x.shape, 1)
        xv = jnp.where(cols < _SIZE, x, 0.0)
        s_rows = jnp.sum(xv, axis=1, keepdims=True)
        partial = jnp.sum(s_rows * mask, axis=(0, 1), keepdims=True)
        sel_hi = ((cols == t) & (t >= _TILE_LIM)).astype(jnp.float32)
        o_t_hi = jnp.sum(xv * sel_hi, axis=1, keepdims=True)
        corr_hi = jnp.sum(o_t_hi * mask, axis=(0, 1), keepdims=True)
        out_ref[0, 0] = -_SV * partial + (_SV - _CONF) * corr_hi


def _tc_partials(output, t32):
    rb = _S_TC // _ROW_BLOCKS
    return pl.pallas_call(
        _tc_body,
        grid=(_ROW_BLOCKS, _NUM_BLOCKS),
        in_specs=[
            pl.BlockSpec((rb, _BLOCK_W), lambda r, k: (r, k)),
            pl.BlockSpec((rb, 1), lambda r, k: (r, 0)),
        ],
        out_specs=pl.BlockSpec((1, 1, 1, 1), lambda r, k: (r, k, 0, 0)),
        out_shape=jax.ShapeDtypeStruct((_ROW_BLOCKS, _NUM_BLOCKS, 1, 1),
                                       jnp.float32),
        compiler_params=pltpu.CompilerParams(
            dimension_semantics=("parallel", "parallel"),
        ),
    )(output, t32)


def _sc_gather_body(x_hbm, tgt_hbm, out_hbm, t_v, tile_v, obuf_v, sem):
    wid = lax.axis_index("s") * _NC + lax.axis_index("c")
    base = wid * _ROWS_PER_W
    pltpu.sync_copy(tgt_hbm.at[pl.ds(base, _ROWS_PER_W)], t_v)
    copies = []
    scalars = []
    for c in range(_VECS_PER_W):
        t16 = t_v[pl.ds(c * 16, 16)]
        for l in range(16):
            r = c * 16 + l
            t_r = t16[l]
            tile = jnp.minimum(lax.shift_right_logical(t_r, 7),
                               _LAST_FULL_TILE)
            col0 = pl.multiple_of(lax.shift_left(tile, 7), 128)
            row0 = pl.multiple_of(base + 8 * (r // 8), 8)
            scalars.append(t_r)
            copies.append(pltpu.async_copy(
                x_hbm.at[pl.ds(row0, 8), pl.ds(col0, 128)],
                tile_v.at[r], sem))
    for cp in copies:
        cp.wait()
    for r in range(_ROWS_PER_W):
        t_r = scalars[r]
        a = jnp.bitwise_and(t_r, 127) - jnp.bitwise_and(t_r, 15)
        obuf_v[r] = tile_v[r, r % 8, pl.ds(a, 16)]
    pltpu.sync_copy(obuf_v, out_hbm.at[pl.ds(base, _ROWS_PER_W)])


def _sc_gather(output, t32):
    tgt = t32.reshape(_N)
    mesh = plsc.VectorSubcoreMesh(core_axis_name="c", subcore_axis_name="s")
    f = functools.partial(
        pl.kernel,
        mesh=mesh,
        out_type=jax.ShapeDtypeStruct((_N, 16), jnp.float32),
        scratch_types=[
            pltpu.VMEM((_ROWS_PER_W,), jnp.int32),
            pltpu.VMEM((_ROWS_PER_W, 8, 128), jnp.float32),
            pltpu.VMEM((_ROWS_PER_W, 16), jnp.float32),
            pltpu.SemaphoreType.DMA,
        ],
    )(_sc_gather_body)
    return f(output, tgt)


def _sc_stream_body(x_hbm, out_hbm, buf_v, vout_v, sem):
    wid = lax.axis_index("s") * _NC + lax.axis_index("c")
    for j in range(_STRIPES_PER_TEC):
        row0 = pl.multiple_of(
            _S_TC + (wid * _STRIPES_PER_TEC + j) * 8, 8)
        accs = tuple(jnp.zeros((16,), jnp.float32) for _ in range(8))

        def chunk_body(i, accs):
            c0 = pl.multiple_of(i * _CW, 128)
            pltpu.async_copy(
                x_hbm.at[pl.ds(row0, 8), pl.ds(c0, _CW)], buf_v, sem
            ).wait()

            def vec_body(v, accs):
                o = v * 16
                return tuple(accs[s] + buf_v[s, pl.ds(o, 16)]
                             for s in range(8))

            return lax.fori_loop(0, _CW // 16, vec_body, accs)

        accs = lax.fori_loop(0, _NFULL, chunk_body, accs)

        c0 = pl.multiple_of(_NFULL * _CW, 128)
        pltpu.async_copy(
            x_hbm.at[pl.ds(row0, 8), pl.ds(c0, _CREM)],
            buf_v.at[:, pl.ds(0, _CREM)], sem).wait()

        def rem_body(v, accs):
            o = v * 16
            return tuple(accs[s] + buf_v[s, pl.ds(o, 16)] for s in range(8))

        accs = lax.fori_loop(0, _CREM // 16, rem_body, accs)
        for s in range(8):
            vout_v[j * 8 + s] = accs[s]

    pltpu.sync_copy(vout_v, out_hbm.at[pl.ds(wid * _ROWS_PER_TEC,
                                             _ROWS_PER_TEC)])


def _sc_stream(output):
    mesh = plsc.VectorSubcoreMesh(core_axis_name="c", subcore_axis_name="s")
    f = functools.partial(
        pl.kernel,
        mesh=mesh,
        out_type=jax.ShapeDtypeStruct((_SC_ROWS, 16), jnp.float32),
        scratch_types=[
            pltpu.VMEM((8, _CW), jnp.float32),
            pltpu.VMEM((_ROWS_PER_TEC, 16), jnp.float32),
            pltpu.SemaphoreType.DMA,
        ],
    )(_sc_stream_body)
    return f(output)


def _comb_body(scs_ref, tail_ref, col0_ref, g_ref, t_ref, out_ref):
    t_all = t_ref[...]
    g_all = g_ref[...]
    mask_all = (t_all != _PADDING_IDX).astype(jnp.float32)
    lanes_all = jax.lax.broadcasted_iota(jnp.int32, g_all.shape, 1)
    sel_all = (lanes_all == t_all % 16).astype(jnp.float32)
    o_t_lo_all = (jnp.sum(g_all * sel_all, axis=1, keepdims=True)
                  * (t_all < _TILE_LIM).astype(jnp.float32))
    corr_tc = jnp.sum((o_t_lo_all * mask_all)[:_S_TC, :],
                      axis=(0, 1), keepdims=True)
    t = t_all[_S_TC:, :]
    mask = mask_all[_S_TC:, :]
    scs = jnp.sum(scs_ref[...], axis=1, keepdims=True)
    tail = tail_ref[...][_S_TC:, :]
    cols = _TILE_LIM + jax.lax.broadcasted_iota(jnp.int32, tail.shape, 1)
    tailv = jnp.where(cols < _SIZE, tail, 0.0)
    tail_sum = jnp.sum(tailv, axis=1, keepdims=True)
    o0 = col0_ref[...][_S_TC:, 0:1]
    o_t_lo = o_t_lo_all[_S_TC:, :]
    sel_hi = (cols == t).astype(jnp.float32)
    o_t_hi = jnp.sum(tailv * sel_hi, axis=1, keepdims=True)
    o_t = o_t_lo + o_t_hi
    s_all = scs + tail_sum - o0 - o_t
    loss = _C0 - _SV * s_all - _CONF * o_t
    out_ref[...] = (jnp.sum(loss * mask, axis=(0, 1), keepdims=True)
                    + (_SV - _CONF) * corr_tc)


def _combine(scs, output, gathered, t32):
    return pl.pallas_call(
        _comb_body,
        grid=(1,),
        in_specs=[
            pl.BlockSpec((_SC_ROWS, 16), lambda k: (0, 0)),
            pl.BlockSpec((_N, 128), lambda k: (0, _TILE_LIM // 128)),
            pl.BlockSpec((_N, 128), lambda k: (0, 0)),
            pl.BlockSpec((_N, 16), lambda k: (0, 0)),
            pl.BlockSpec((_N, 1), lambda k: (0, 0)),
        ],
        out_specs=pl.BlockSpec((1, 1), lambda k: (0, 0)),
        out_shape=jax.ShapeDtypeStruct((1, 1), jnp.float32),
    )(scs, output, output, gathered, t32)


@jax.jit
def kernel(output, target):
    t32 = target.astype(jnp.int32)
    gathered = _sc_gather(output, t32)
    scs = _sc_stream(output)
    tc = _tc_partials(output, t32)
    comb = _combine(scs, output, gathered, t32)
    return jnp.sum(tc) + comb[0, 0]


# speedup vs baseline: 3.8017x; 3.7982x over previous
import functools
import math

import jax
import jax.numpy as jnp
from jax import lax
from jax.experimental import pallas as pl
from jax.experimental.pallas import tpu as pltpu
from jax.experimental.pallas import tpu_sc as plsc

_SIZE = 100000
_PADDING_IDX = 0
_LABEL_SMOOTHING = 0.1
_SV = _LABEL_SMOOTHING / (_SIZE - 2)
_CONF = 1.0 - _LABEL_SMOOTHING
_C0 = (_SIZE - 2) * _SV * math.log(_SV) + _CONF * math.log(_CONF)

_N = 1024
_BLOCK_V = 2000
_NUM_BLOCKS = _SIZE // _BLOCK_V

_NC = 2
_NS = 16
_NW = _NC * _NS
_ROWS_PER_W = _N // _NW
_VECS_PER_W = _ROWS_PER_W // 16


def _tc_body(x_ref, t_ref, out_ref):
    k = pl.program_id(0)
    x = x_ref[...]
    t = t_ref[...]
    mask = (t != _PADDING_IDX).astype(jnp.float32)

    @pl.when(k == 0)
    def _first():
        csum = jnp.sum(x, axis=0, keepdims=True) - x[0:1, :]
        partial = jnp.sum(csum * mask, axis=(0, 1), keepdims=True)
        cnt = jnp.sum(mask, axis=(0, 1), keepdims=True)
        out_ref[0] = _C0 * cnt - _SV * partial

    @pl.when(k != 0)
    def _rest():
        csum = jnp.sum(x, axis=0, keepdims=True)
        partial = jnp.sum(csum * mask, axis=(0, 1), keepdims=True)
        out_ref[0] = -_SV * partial


def _tc_partials(xt, trow):
    return pl.pallas_call(
        _tc_body,
        grid=(_NUM_BLOCKS,),
        in_specs=[
            pl.BlockSpec((_BLOCK_V, _N), lambda k: (k, 0)),
            pl.BlockSpec((1, _N), lambda k: (0, 0)),
        ],
        out_specs=pl.BlockSpec((1, 1, 1), lambda k: (k, 0, 0)),
        out_shape=jax.ShapeDtypeStruct((_NUM_BLOCKS, 1, 1), jnp.float32),
        compiler_params=pltpu.CompilerParams(
            dimension_semantics=("parallel",),
        ),
    )(xt, trow)


def _sc_gather_body(x_hbm, tgt_hbm, out_hbm, t_v, tile_v, obuf_v, sem):
    wid = lax.axis_index("s") * _NC + lax.axis_index("c")
    base = wid * _ROWS_PER_W
    pltpu.sync_copy(tgt_hbm.at[pl.ds(base, _ROWS_PER_W)], t_v)
    copies = []
    scalars = []
    for c in range(_VECS_PER_W):
        t16 = t_v[pl.ds(c * 16, 16)]
        for l in range(16):
            r = c * 16 + l
            t_r = t16[l]
            trow0 = pl.multiple_of(t_r - jnp.bitwise_and(t_r, 7), 8)
            col0 = pl.multiple_of((base + r) - (base + r) % 128, 128)
            scalars.append(t_r)
            copies.append(pltpu.async_copy(
                x_hbm.at[pl.ds(trow0, 8), pl.ds(col0, 128)],
                tile_v.at[r], sem))
    for cp in copies:
        cp.wait()
    for r in range(_ROWS_PER_W):
        t_r = scalars[r]
        s_dyn = jnp.bitwise_and(t_r, 7)
        a = ((base + r) % 128) - ((base + r) % 16)
        obuf_v[r] = tile_v[r, s_dyn, pl.ds(a, 16)]
    pltpu.sync_copy(obuf_v, out_hbm.at[pl.ds(base, _ROWS_PER_W)])


def _sc_gather(xt, t32):
    tgt = t32.reshape(_N)
    mesh = plsc.VectorSubcoreMesh(core_axis_name="c", subcore_axis_name="s")
    f = functools.partial(
        pl.kernel,
        mesh=mesh,
        out_type=jax.ShapeDtypeStruct((_N, 16), jnp.float32),
        scratch_types=[
            pltpu.VMEM((_ROWS_PER_W,), jnp.int32),
            pltpu.VMEM((_ROWS_PER_W, 8, 128), jnp.float32),
            pltpu.VMEM((_ROWS_PER_W, 16), jnp.float32),
            pltpu.SemaphoreType.DMA,
        ],
    )(_sc_gather_body)
    return f(xt, tgt)


def _comb_body(g_ref, t_ref, out_ref):
    t = t_ref[...]
    mask = (t != _PADDING_IDX).astype(jnp.float32)
    g = g_ref[...]
    rows = jax.lax.broadcasted_iota(jnp.int32, g.shape, 0)
    lanes = jax.lax.broadcasted_iota(jnp.int32, g.shape, 1)
    sel = (lanes == rows % 16).astype(jnp.float32)
    o_t = jnp.sum(g * sel, axis=1, keepdims=True)
    out_ref[...] = (_SV - _CONF) * jnp.sum(
        o_t * mask, axis=(0, 1), keepdims=True)


def _combine(gathered, t32):
    return pl.pallas_call(
        _comb_body,
        grid=(1,),
        in_specs=[
            pl.BlockSpec((_N, 16), lambda k: (0, 0)),
            pl.BlockSpec((_N, 1), lambda k: (0, 0)),
        ],
        out_specs=pl.BlockSpec((1, 1), lambda k: (0, 0)),
        out_shape=jax.ShapeDtypeStruct((1, 1), jnp.float32),
    )(gathered, t32)


@jax.jit
def kernel(output, target):
    t32 = target.astype(jnp.int32)
    xt = output.T
    trow = t32.reshape(1, _N)
    gathered = _sc_gather(xt, t32)
    tc = _tc_partials(xt, trow)
    comb = _combine(gathered, t32)
    return jnp.sum(tc) + comb[0, 0]
